# Initial kernel scaffold; baseline (speedup 1.0000x reference)
import jax
import jax.numpy as jnp
from jax.experimental import pallas as pl


def kernel(feat, edge_index, radius, exp, eps):
    raise NotImplementedError("write your pallas kernel here")


# speedup vs baseline: 4.3604x; 4.3604x over previous
import functools

import jax
import jax.numpy as jnp
from jax import lax
from jax.experimental import pallas as pl
from jax.experimental.pallas import tpu as pltpu
from jax.experimental.pallas import tpu_sc as plsc

NC = 2
NS = 16
NW = NC * NS
CHUNK = 128


def _sc_scatter(feat, src3, dst3, zeros, n_pad, q, d):
    zrows = n_pad // NS

    mesh = plsc.VectorSubcoreMesh(core_axis_name="c", subcore_axis_name="s")

    @functools.partial(
        pl.kernel,
        out_type=jax.ShapeDtypeStruct((NC, n_pad, d), jnp.float32),
        mesh=mesh,
        scratch_types=[
            pltpu.VMEM((q, CHUNK), jnp.int32),
            pltpu.VMEM((q, CHUNK), jnp.int32),
            pltpu.VMEM((CHUNK, d), jnp.float32),
            pltpu.VMEM_SHARED((n_pad, d), jnp.float32),
            pltpu.SemaphoreType.DMA,
        ],
    )
    def k(feat_h, src_h, dst_h, zeros_h, out_h, src_v, dst_v, rows_v, acc, sem):
        c = lax.axis_index("c")
        s = lax.axis_index("s")
        wid = c * NS + s
        pltpu.sync_copy(src_h.at[wid], src_v)
        pltpu.sync_copy(dst_h.at[wid], dst_v)
        pltpu.sync_copy(zeros_h.at[pl.ds(s * zrows, zrows)],
                        acc.at[pl.ds(s * zrows, zrows)])
        plsc.subcore_barrier()

        @pl.loop(0, q)
        def _(j):
            pltpu.async_copy(feat_h.at[src_v.at[j]], rows_v, sem).wait()
            pltpu.sync_copy(rows_v, acc.at[dst_v.at[j]], add=True)

        plsc.subcore_barrier()
        pltpu.sync_copy(acc.at[pl.ds(s * zrows, zrows)],
                        out_h.at[c, pl.ds(s * zrows, zrows)])

    return k(feat, src3, dst3, zeros)


def _tc_combine(eps, feat, p0, p1, rb):
    n, d = feat.shape

    def body(eps_ref, feat_ref, p0_ref, p1_ref, out_ref):
        out_ref[...] = ((1.0 + eps_ref[0]) * feat_ref[...]
                        + p0_ref[...] + p1_ref[...])

    return pl.pallas_call(
        body,
        out_shape=jax.ShapeDtypeStruct((n, d), jnp.float32),
        grid=(n // rb,),
        in_specs=[
            pl.BlockSpec(memory_space=pltpu.SMEM),
            pl.BlockSpec((rb, d), lambda i: (i, 0)),
            pl.BlockSpec((rb, d), lambda i: (i, 0)),
            pl.BlockSpec((rb, d), lambda i: (i, 0)),
        ],
        out_specs=pl.BlockSpec((rb, d), lambda i: (i, 0)),
    )(eps, feat, p0, p1)


def kernel(feat, edge_index, radius, exp, eps):
    del radius, exp
    n, d = feat.shape
    e = edge_index.shape[1]

    q = -(-e // (NW * CHUNK))
    e_pad = NW * q * CHUNK
    n_pad = -(-(n + 1) // (NS * 8)) * (NS * 8)

    pad = e_pad - e
    src = jnp.concatenate([edge_index[0], jnp.zeros((pad,), jnp.int32)])
    dst = jnp.concatenate([edge_index[1], jnp.full((pad,), n, jnp.int32)])
    src3 = src.reshape(NW, q, CHUNK)
    dst3 = dst.reshape(NW, q, CHUNK)
    zeros = jnp.zeros((n_pad, d), jnp.float32)

    partials = _sc_scatter(feat, src3, dst3, zeros, n_pad, q, d)
    return _tc_combine(eps, feat, partials[0, :n], partials[1, :n], rb=1000)
